# baseline (speedup 1.0000x reference)
import functools

import jax
import jax.numpy as jnp
from jax import lax
from jax.experimental import pallas as pl
from jax.experimental.pallas import tpu as pltpu
from jax.experimental.pallas import tpu_sc as plsc

B, T, D = 2, 8192, 768
H = 12
DH = D // H
K = 64
SCALE = DH ** -0.5
BT = 512
NT = T // BT

NC, NS = 2, 16
NW = NC * NS
RPW = (B * H * K) // NW
PADL = 128

_DOT = functools.partial(lax.dot_general, preferred_element_type=jnp.float32,
                         precision=lax.Precision.DEFAULT)


def _scores_body(x_ref, wk_ref, wbd_ref, s_ref):
    xb = x_ref[0]
    kb = _DOT(xb, wk_ref[...], (((1,), (1,)), ((), ())))
    s_ref[0] = _DOT(wbd_ref[...], kb, (((1,), (1,)), ((), ())))


def _topk_body(s_ref, idx_ref, scratch):
    scratch[...] = s_ref[...].reshape(B * H, T)
    iota_t = lax.broadcasted_iota(jnp.int32, (B * H, T), 1)
    rows = lax.broadcasted_iota(jnp.int32, (B * H, 1), 0)
    base = (rows // H) * T
    col = lax.broadcasted_iota(jnp.int32, (B * H, K), 1)

    def body(i, acc):
        s = scratch[...]
        m = jnp.max(s, axis=1, keepdims=True)
        idx = jnp.min(jnp.where(s == m, iota_t, T), axis=1, keepdims=True)
        scratch[...] = jnp.where(iota_t == idx, -jnp.inf, s)
        return jnp.where(col == i, idx + base, acc)

    idx_ref[...] = lax.fori_loop(0, K, body, jnp.zeros((B * H, K), jnp.int32))


def _sc_gather_body(x_ref, idx_ref, out_ref, idx_v, rows_v, sem):
    wid = lax.axis_index("s") * NC + lax.axis_index("c")
    base = wid * RPW
    pltpu.sync_copy(idx_ref.at[pl.ds(base, RPW)], idx_v)
    pltpu.async_copy(x_ref.at[idx_v], rows_v, sem).wait()
    pltpu.sync_copy(rows_v, out_ref.at[pl.ds(base, RPW)])


def _attn_body(x_ref, xs_ref, wq_ref, wk_ref, wv_ref, wot_ref,
               out_ref, kbd_s, vbd_s):
    t = pl.program_id(1)

    @pl.when(t == 0)
    def _build():
        xs = xs_ref[0]
        kf = _DOT(xs, wk_ref[...], (((1,), (1,)), ((), ())))
        vf = _DOT(xs, wv_ref[...], (((1,), (1,)), ((), ())))
        row = lax.broadcasted_iota(jnp.int32, (H * K, D), 0)
        colc = lax.broadcasted_iota(jnp.int32, (H * K, D), 1)
        mask = (row // K) == (colc // DH)
        kbd_s[...] = jnp.where(mask, kf, 0.0)
        vbd_s[:, :D] = jnp.where(mask, vf, 0.0)
        colo = lax.broadcasted_iota(jnp.int32, (H * K, PADL), 1)
        rowo = lax.broadcasted_iota(jnp.int32, (H * K, PADL), 0)
        vbd_s[:, D:] = jnp.where((rowo // K) == colo, 1.0, 0.0)

    qb = _DOT(x_ref[0], wq_ref[...], (((1,), (1,)), ((), ()))) * SCALE
    logits = _DOT(qb, kbd_s[...], (((1,), (1,)), ((), ())))
    p = jnp.exp(logits)
    oh_aug = _DOT(p, vbd_s[...], (((1,), (0,)), ((), ())))
    s_rep = jnp.repeat(oh_aug[:, D:D + H], DH, axis=1)
    oh = oh_aug[:, :D] / s_rep
    out_ref[0] = _DOT(oh, wot_ref[...], (((1,), (0,)), ((), ())))


def kernel(x, Wq, Wk, Wv, Wo, w_idx):
    f32 = jnp.float32
    w_bd = jnp.kron(jnp.eye(H, dtype=f32), w_idx.reshape(1, DH))

    scores = pl.pallas_call(
        _scores_body,
        grid=(B, NT),
        in_specs=[
            pl.BlockSpec((1, BT, D), lambda b, t: (b, t, 0)),
            pl.BlockSpec((D, D), lambda b, t: (0, 0)),
            pl.BlockSpec((H, D), lambda b, t: (0, 0)),
        ],
        out_specs=pl.BlockSpec((1, H, BT), lambda b, t: (b, 0, t)),
        out_shape=jax.ShapeDtypeStruct((B, H, T), f32),
        compiler_params=pltpu.CompilerParams(
            dimension_semantics=("parallel", "parallel")),
    )(x, Wk, w_bd)

    flat_idx = pl.pallas_call(
        _topk_body,
        out_shape=jax.ShapeDtypeStruct((B * H, K), jnp.int32),
        scratch_shapes=[pltpu.VMEM((B * H, T), f32)],
    )(scores)

    sc_gather = functools.partial(
        pl.kernel,
        mesh=plsc.VectorSubcoreMesh(core_axis_name="c", subcore_axis_name="s"),
        out_type=jax.ShapeDtypeStruct((B * H * K, D), f32),
        scratch_types=[
            pltpu.VMEM((RPW,), jnp.int32),
            pltpu.VMEM((RPW, D), f32),
            pltpu.SemaphoreType.DMA,
        ],
    )(_sc_gather_body)
    x_sel = sc_gather(x.reshape(B * T, D), flat_idx.reshape(B * H * K))

    wot = Wo.T

    out = pl.pallas_call(
        _attn_body,
        grid=(B, NT),
        in_specs=[
            pl.BlockSpec((1, BT, D), lambda b, t: (b, t, 0)),
            pl.BlockSpec((1, H * K, D), lambda b, t: (b, 0, 0)),
            pl.BlockSpec((D, D), lambda b, t: (0, 0)),
            pl.BlockSpec((D, D), lambda b, t: (0, 0)),
            pl.BlockSpec((D, D), lambda b, t: (0, 0)),
            pl.BlockSpec((D, D), lambda b, t: (0, 0)),
        ],
        out_specs=pl.BlockSpec((1, BT, D), lambda b, t: (b, t, 0)),
        out_shape=jax.ShapeDtypeStruct((B, T, D), f32),
        scratch_shapes=[pltpu.VMEM((H * K, D), f32),
                        pltpu.VMEM((H * K, D + PADL), f32)],
        compiler_params=pltpu.CompilerParams(
            dimension_semantics=("arbitrary", "arbitrary")),
    )(x, x_sel.reshape(B, H * K, D), Wq, Wk, Wv, wot)

    return out


# speedup vs baseline: 1.1856x; 1.1856x over previous
import functools

import jax
import jax.numpy as jnp
from jax import lax
from jax.experimental import pallas as pl
from jax.experimental.pallas import tpu as pltpu
from jax.experimental.pallas import tpu_sc as plsc

B, T, D = 2, 8192, 768
H = 12
DH = D // H
K = 64
SCALE = DH ** -0.5
BT = 1024
NT = T // BT

NC, NS = 2, 16
NW = NC * NS
RPW = (B * H * K) // NW
PADL = 128

_DOT = functools.partial(lax.dot_general, preferred_element_type=jnp.float32,
                         precision=lax.Precision.DEFAULT)


def _scores_body(x_ref, wk_ref, wbd_ref, s_ref):
    xb = x_ref[0]
    kb = _DOT(xb, wk_ref[...], (((1,), (1,)), ((), ())))
    s_ref[0] = _DOT(wbd_ref[...], kb, (((1,), (1,)), ((), ())))


def _topk_body(s_ref, idx_ref, scratch):
    scratch[...] = s_ref[...].reshape(B * H, T)
    iota_t = lax.broadcasted_iota(jnp.int32, (B * H, T), 1)
    rows = lax.broadcasted_iota(jnp.int32, (B * H, 1), 0)
    base = (rows // H) * T
    col = lax.broadcasted_iota(jnp.int32, (B * H, K), 1)

    def body(i, acc):
        s = scratch[...]
        m = jnp.max(s, axis=1, keepdims=True)
        idx = jnp.min(jnp.where(s == m, iota_t, T), axis=1, keepdims=True)
        scratch[...] = jnp.where(iota_t == idx, -jnp.inf, s)
        return jnp.where(col == i, idx + base, acc)

    idx_ref[...] = lax.fori_loop(0, K, body, jnp.zeros((B * H, K), jnp.int32))


def _sc_gather_body(x_ref, idx_ref, out_ref, idx_v, rows_v, sem):
    wid = lax.axis_index("s") * NC + lax.axis_index("c")
    base = wid * RPW
    pltpu.sync_copy(idx_ref.at[pl.ds(base, RPW)], idx_v)
    pltpu.async_copy(x_ref.at[idx_v], rows_v, sem).wait()
    pltpu.sync_copy(rows_v, out_ref.at[pl.ds(base, RPW)])


def _attn_body(x_ref, xs_ref, wq_ref, wk_ref, wv_ref, ones_ref, wot_ref,
               out_ref, kbd_s, vbd_s):
    t = pl.program_id(1)

    @pl.when(t == 0)
    def _build():
        xs = xs_ref[0]
        kf = _DOT(xs, wk_ref[...], (((1,), (1,)), ((), ())))
        vf = _DOT(xs, wv_ref[...], (((1,), (1,)), ((), ())))
        row = lax.broadcasted_iota(jnp.int32, (H * K, D), 0)
        colc = lax.broadcasted_iota(jnp.int32, (H * K, D), 1)
        mask = (row // K) == (colc // DH)
        kbd_s[...] = jnp.where(mask, kf, 0.0)
        vbd_s[...] = jnp.where(mask, vf, 0.0)

    qb = _DOT(x_ref[0], wq_ref[...], (((1,), (1,)), ((), ()))) * SCALE
    logits = _DOT(qb, kbd_s[...], (((1,), (1,)), ((), ())))
    p = jnp.exp(logits)
    s_rep = _DOT(p, ones_ref[...], (((1,), (0,)), ((), ())))
    oh = _DOT(p, vbd_s[...], (((1,), (0,)), ((), ()))) / s_rep
    out_ref[0] = _DOT(oh, wot_ref[...], (((1,), (0,)), ((), ())))


def kernel(x, Wq, Wk, Wv, Wo, w_idx):
    f32 = jnp.float32
    w_bd = jnp.kron(jnp.eye(H, dtype=f32), w_idx.reshape(1, DH))
    ones_bd = jnp.kron(jnp.eye(H, dtype=f32), jnp.ones((DH, DH), f32))

    scores = pl.pallas_call(
        _scores_body,
        grid=(B, NT),
        in_specs=[
            pl.BlockSpec((1, BT, D), lambda b, t: (b, t, 0)),
            pl.BlockSpec((D, D), lambda b, t: (0, 0)),
            pl.BlockSpec((H, D), lambda b, t: (0, 0)),
        ],
        out_specs=pl.BlockSpec((1, H, BT), lambda b, t: (b, 0, t)),
        out_shape=jax.ShapeDtypeStruct((B, H, T), f32),
        compiler_params=pltpu.CompilerParams(
            dimension_semantics=("parallel", "parallel")),
    )(x, Wk, w_bd)

    flat_idx = pl.pallas_call(
        _topk_body,
        out_shape=jax.ShapeDtypeStruct((B * H, K), jnp.int32),
        scratch_shapes=[pltpu.VMEM((B * H, T), f32)],
    )(scores)

    sc_gather = functools.partial(
        pl.kernel,
        mesh=plsc.VectorSubcoreMesh(core_axis_name="c", subcore_axis_name="s"),
        out_type=jax.ShapeDtypeStruct((B * H * K, D), f32),
        scratch_types=[
            pltpu.VMEM((RPW,), jnp.int32),
            pltpu.VMEM((RPW, D), f32),
            pltpu.SemaphoreType.DMA,
        ],
    )(_sc_gather_body)
    x_sel = sc_gather(x.reshape(B * T, D), flat_idx.reshape(B * H * K))

    wot = Wo.T

    out = pl.pallas_call(
        _attn_body,
        grid=(B, NT),
        in_specs=[
            pl.BlockSpec((1, BT, D), lambda b, t: (b, t, 0)),
            pl.BlockSpec((1, H * K, D), lambda b, t: (b, 0, 0)),
            pl.BlockSpec((D, D), lambda b, t: (0, 0)),
            pl.BlockSpec((D, D), lambda b, t: (0, 0)),
            pl.BlockSpec((D, D), lambda b, t: (0, 0)),
            pl.BlockSpec((D, D), lambda b, t: (0, 0)),
            pl.BlockSpec((H * K, D), lambda b, t: (0, 0)),
        ],
        out_specs=pl.BlockSpec((1, BT, D), lambda b, t: (b, t, 0)),
        out_shape=jax.ShapeDtypeStruct((B, T, D), f32),
        scratch_shapes=[pltpu.VMEM((H * K, D), f32),
                        pltpu.VMEM((H * K, D), f32)],
        compiler_params=pltpu.CompilerParams(
            dimension_semantics=("arbitrary", "arbitrary")),
    )(x, x_sel.reshape(B, H * K, D), Wq, Wk, Wv, ones_bd, wot)

    return out


# speedup vs baseline: 1.2847x; 1.0836x over previous
import functools

import jax
import jax.numpy as jnp
from jax import lax
from jax.experimental import pallas as pl
from jax.experimental.pallas import tpu as pltpu
from jax.experimental.pallas import tpu_sc as plsc

B, T, D = 2, 8192, 768
H = 12
DH = D // H
K = 64
SCALE = DH ** -0.5
BT = 1024
NT = T // BT

NC, NS = 2, 16
NW = NC * NS
RPW = (B * H * K) // NW
PADL = 128

_DOT = functools.partial(lax.dot_general, preferred_element_type=jnp.float32,
                         precision=lax.Precision.DEFAULT)


def _scores_body(x_ref, wk_ref, wbd_ref, s_ref):
    xb = x_ref[0]
    kb = _DOT(xb, wk_ref[...], (((1,), (1,)), ((), ())))
    s_ref[0] = _DOT(wbd_ref[...], kb, (((1,), (1,)), ((), ())))


def _topk_body(s_ref, idx_ref, scratch):
    scratch[...] = s_ref[...].reshape(B * H, T)
    iota_t = lax.broadcasted_iota(jnp.int32, (B * H, T), 1)
    rows = lax.broadcasted_iota(jnp.int32, (B * H, 1), 0)
    base = (rows // H) * T
    col = lax.broadcasted_iota(jnp.int32, (B * H, K), 1)

    def body(i, acc):
        s = scratch[...]
        m = jnp.max(s, axis=1, keepdims=True)
        idx = jnp.min(jnp.where(s == m, iota_t, T), axis=1, keepdims=True)
        scratch[...] = jnp.where(iota_t == idx, -jnp.inf, s)
        return jnp.where(col == i, idx + base, acc)

    idx_ref[...] = lax.fori_loop(0, K, body, jnp.zeros((B * H, K), jnp.int32))


def _sc_gather_body(x_ref, idx_ref, out_ref, idx_v, rows_v, sem):
    wid = lax.axis_index("s") * NC + lax.axis_index("c")
    base = wid * RPW
    pltpu.sync_copy(idx_ref.at[pl.ds(base, RPW)], idx_v)
    pltpu.async_copy(x_ref.at[idx_v], rows_v, sem).wait()
    pltpu.sync_copy(rows_v, out_ref.at[pl.ds(base, RPW)])


def _attn_body(x_ref, xs_ref, wq_ref, wk_ref, wv_ref, ones_ref, wot_ref,
               out_ref, kbd_s, m_s, vbd_s):
    t = pl.program_id(1)

    @pl.when(t == 0)
    def _build():
        xs = xs_ref[0]
        kf = _DOT(xs, wk_ref[...], (((1,), (1,)), ((), ())))
        vf = _DOT(xs, wv_ref[...], (((1,), (1,)), ((), ())))
        row = lax.broadcasted_iota(jnp.int32, (H * K, D), 0)
        colc = lax.broadcasted_iota(jnp.int32, (H * K, D), 1)
        mask = (row // K) == (colc // DH)
        kbd_s[...] = jnp.where(mask, kf * SCALE, 0.0)
        vbd_s[...] = jnp.where(mask, vf, 0.0)
        m_s[...] = _DOT(kbd_s[...], wq_ref[...], (((1,), (0,)), ((), ())))

    logits = _DOT(x_ref[0], m_s[...], (((1,), (1,)), ((), ())))
    p = jnp.exp(logits)
    s_rep = _DOT(p, ones_ref[...], (((1,), (0,)), ((), ())))
    oh = _DOT(p, vbd_s[...], (((1,), (0,)), ((), ()))) / s_rep
    out_ref[0] = _DOT(oh, wot_ref[...], (((1,), (0,)), ((), ())))


def kernel(x, Wq, Wk, Wv, Wo, w_idx):
    f32 = jnp.float32
    w_bd = jnp.kron(jnp.eye(H, dtype=f32), w_idx.reshape(1, DH))
    ones_bd = jnp.kron(jnp.eye(H, dtype=f32), jnp.ones((DH, DH), f32))

    scores = pl.pallas_call(
        _scores_body,
        grid=(B, NT),
        in_specs=[
            pl.BlockSpec((1, BT, D), lambda b, t: (b, t, 0)),
            pl.BlockSpec((D, D), lambda b, t: (0, 0)),
            pl.BlockSpec((H, D), lambda b, t: (0, 0)),
        ],
        out_specs=pl.BlockSpec((1, H, BT), lambda b, t: (b, 0, t)),
        out_shape=jax.ShapeDtypeStruct((B, H, T), f32),
        compiler_params=pltpu.CompilerParams(
            dimension_semantics=("parallel", "parallel")),
    )(x, Wk, w_bd)

    flat_idx = pl.pallas_call(
        _topk_body,
        out_shape=jax.ShapeDtypeStruct((B * H, K), jnp.int32),
        scratch_shapes=[pltpu.VMEM((B * H, T), f32)],
    )(scores)

    sc_gather = functools.partial(
        pl.kernel,
        mesh=plsc.VectorSubcoreMesh(core_axis_name="c", subcore_axis_name="s"),
        out_type=jax.ShapeDtypeStruct((B * H * K, D), f32),
        scratch_types=[
            pltpu.VMEM((RPW,), jnp.int32),
            pltpu.VMEM((RPW, D), f32),
            pltpu.SemaphoreType.DMA,
        ],
    )(_sc_gather_body)
    x_sel = sc_gather(x.reshape(B * T, D), flat_idx.reshape(B * H * K))

    wot = Wo.T

    out = pl.pallas_call(
        _attn_body,
        grid=(B, NT),
        in_specs=[
            pl.BlockSpec((1, BT, D), lambda b, t: (b, t, 0)),
            pl.BlockSpec((1, H * K, D), lambda b, t: (b, 0, 0)),
            pl.BlockSpec((D, D), lambda b, t: (0, 0)),
            pl.BlockSpec((D, D), lambda b, t: (0, 0)),
            pl.BlockSpec((D, D), lambda b, t: (0, 0)),
            pl.BlockSpec((D, D), lambda b, t: (0, 0)),
            pl.BlockSpec((H * K, D), lambda b, t: (0, 0)),
        ],
        out_specs=pl.BlockSpec((1, BT, D), lambda b, t: (b, t, 0)),
        out_shape=jax.ShapeDtypeStruct((B, T, D), f32),
        scratch_shapes=[pltpu.VMEM((H * K, D), f32),
                        pltpu.VMEM((H * K, D), f32),
                        pltpu.VMEM((H * K, D), f32)],
        compiler_params=pltpu.CompilerParams(
            dimension_semantics=("arbitrary", "arbitrary")),
    )(x, x_sel.reshape(B, H * K, D), Wq, Wk, Wv, ones_bd, wot)

    return out


# speedup vs baseline: 1.4082x; 1.0961x over previous
import functools

import jax
import jax.numpy as jnp
from jax import lax
from jax.experimental import pallas as pl
from jax.experimental.pallas import tpu as pltpu
from jax.experimental.pallas import tpu_sc as plsc

B, T, D = 2, 8192, 768
H = 12
DH = D // H
K = 64
SCALE = DH ** -0.5
BT = 1024
NT = T // BT

NC, NS = 2, 16
NW = NC * NS
RPW = (B * H * K) // NW
PADL = 128

_DOT = functools.partial(lax.dot_general, preferred_element_type=jnp.float32,
                         precision=lax.Precision.DEFAULT)


def _scores_body(x_ref, wk_ref, wbd_ref, s_ref):
    xb = x_ref[0]
    kb = _DOT(xb, wk_ref[...], (((1,), (1,)), ((), ())))
    s_ref[0] = _DOT(wbd_ref[...], kb, (((1,), (1,)), ((), ())))


def _topk_body(s_ref, idx_ref, scratch):
    scratch[...] = s_ref[...].reshape(B * H, T)
    iota_t = lax.broadcasted_iota(jnp.int32, (B * H, T), 1)
    rows = lax.broadcasted_iota(jnp.int32, (B * H, 1), 0)
    base = (rows // H) * T
    col = lax.broadcasted_iota(jnp.int32, (B * H, K), 1)

    def body(i, acc):
        s = scratch[...]
        m = jnp.max(s, axis=1, keepdims=True)
        idx = jnp.min(jnp.where(s == m, iota_t, T), axis=1, keepdims=True)
        scratch[...] = jnp.where(iota_t == idx, -jnp.inf, s)
        return jnp.where(col == i, idx + base, acc)

    idx_ref[...] = lax.fori_loop(0, K, body, jnp.zeros((B * H, K), jnp.int32))


def _sc_gather_body(x_ref, idx_ref, out_ref, idx_v, rows_v, sem):
    wid = lax.axis_index("s") * NC + lax.axis_index("c")
    base = wid * RPW
    pltpu.sync_copy(idx_ref.at[pl.ds(base, RPW)], idx_v)
    pltpu.async_copy(x_ref.at[idx_v], rows_v, sem).wait()
    pltpu.sync_copy(rows_v, out_ref.at[pl.ds(base, RPW)])


def _attn_body(x_ref, xs_ref, wq_ref, wk_ref, wv_ref, ones_ref, wot_ref,
               out_ref, kbd_s, m_s, vbd_s, n_s):
    t = pl.program_id(1)

    @pl.when(t == 0)
    def _build():
        xs = xs_ref[0]
        kf = _DOT(xs, wk_ref[...], (((1,), (1,)), ((), ())))
        vf = _DOT(xs, wv_ref[...], (((1,), (1,)), ((), ())))
        row = lax.broadcasted_iota(jnp.int32, (H * K, D), 0)
        colc = lax.broadcasted_iota(jnp.int32, (H * K, D), 1)
        mask = (row // K) == (colc // DH)
        kbd_s[...] = jnp.where(mask, kf * SCALE, 0.0)
        vbd_s[...] = jnp.where(mask, vf, 0.0)
        m_s[...] = _DOT(kbd_s[...], wq_ref[...], (((1,), (0,)), ((), ())))
        n_s[...] = _DOT(vbd_s[...], wot_ref[...], (((1,), (0,)), ((), ())))

    logits = _DOT(x_ref[0], m_s[...], (((1,), (1,)), ((), ())))
    p = jnp.exp(logits)
    s_rep = _DOT(p, ones_ref[...], (((1,), (0,)), ((), ())))
    attn = p / s_rep
    out_ref[0] = _DOT(attn, n_s[...], (((1,), (0,)), ((), ())))


def kernel(x, Wq, Wk, Wv, Wo, w_idx):
    f32 = jnp.float32
    w_bd = jnp.kron(jnp.eye(H, dtype=f32), w_idx.reshape(1, DH))
    ones_bd = jnp.kron(jnp.eye(H, dtype=f32), jnp.ones((DH, DH), f32))

    scores = pl.pallas_call(
        _scores_body,
        grid=(B, NT),
        in_specs=[
            pl.BlockSpec((1, BT, D), lambda b, t: (b, t, 0)),
            pl.BlockSpec((D, D), lambda b, t: (0, 0)),
            pl.BlockSpec((H, D), lambda b, t: (0, 0)),
        ],
        out_specs=pl.BlockSpec((1, H, BT), lambda b, t: (b, 0, t)),
        out_shape=jax.ShapeDtypeStruct((B, H, T), f32),
        compiler_params=pltpu.CompilerParams(
            dimension_semantics=("parallel", "parallel")),
    )(x, Wk, w_bd)

    flat_idx = pl.pallas_call(
        _topk_body,
        out_shape=jax.ShapeDtypeStruct((B * H, K), jnp.int32),
        scratch_shapes=[pltpu.VMEM((B * H, T), f32)],
    )(scores)

    sc_gather = functools.partial(
        pl.kernel,
        mesh=plsc.VectorSubcoreMesh(core_axis_name="c", subcore_axis_name="s"),
        out_type=jax.ShapeDtypeStruct((B * H * K, D), f32),
        scratch_types=[
            pltpu.VMEM((RPW,), jnp.int32),
            pltpu.VMEM((RPW, D), f32),
            pltpu.SemaphoreType.DMA,
        ],
    )(_sc_gather_body)
    x_sel = sc_gather(x.reshape(B * T, D), flat_idx.reshape(B * H * K))

    wot = Wo.T

    out = pl.pallas_call(
        _attn_body,
        grid=(B, NT),
        in_specs=[
            pl.BlockSpec((1, BT, D), lambda b, t: (b, t, 0)),
            pl.BlockSpec((1, H * K, D), lambda b, t: (b, 0, 0)),
            pl.BlockSpec((D, D), lambda b, t: (0, 0)),
            pl.BlockSpec((D, D), lambda b, t: (0, 0)),
            pl.BlockSpec((D, D), lambda b, t: (0, 0)),
            pl.BlockSpec((D, D), lambda b, t: (0, 0)),
            pl.BlockSpec((H * K, D), lambda b, t: (0, 0)),
        ],
        out_specs=pl.BlockSpec((1, BT, D), lambda b, t: (b, t, 0)),
        out_shape=jax.ShapeDtypeStruct((B, T, D), f32),
        scratch_shapes=[pltpu.VMEM((H * K, D), f32),
                        pltpu.VMEM((H * K, D), f32),
                        pltpu.VMEM((H * K, D), f32),
                        pltpu.VMEM((H * K, D), f32)],
        compiler_params=pltpu.CompilerParams(
            dimension_semantics=("arbitrary", "arbitrary")),
    )(x, x_sel.reshape(B, H * K, D), Wq, Wk, Wv, ones_bd, wot)

    return out
